# baseline (speedup 1.0000x reference)
import functools

import jax
import jax.numpy as jnp
from jax import lax
from jax.experimental import pallas as pl
from jax.experimental.pallas import tpu as pltpu
from jax.experimental.pallas import tpu_sc as plsc

C = 8
D = 4
V = 1000
B = 16384
L = 16
NC = 2
NS = 16
NW = NC * NS
BPW = B // NW
NV = BPW // L

LN2 = 0.6931471805599453
MIN_LL = -23.025850929940457
_P = [c * LN2 for c in (-2.153433788869514, 3.0475643969438027,
                        -1.051706410441485, 0.15822203552974923)]
_P0 = _P[0] - 127.0 * LN2


def _ln(s):
    bits = lax.bitcast_convert_type(s, jnp.int32)
    ef = lax.shift_right_arithmetic(bits, 23).astype(jnp.float32)
    mbits = (bits & 0x007FFFFF) | 0x3F800000
    m = lax.bitcast_convert_type(mbits, jnp.float32)
    p = _P[3]
    p = p * m + _P[2]
    p = p * m + _P[1]
    p = p * m + _P0
    return ef * LN2 + p


def _sc_body(esp_hbm, ems_hbm, er_hbm, ann_hbm, atr_hbm, conf_hbm, out_hbm,
             esp_v, ems_v, er_v, ann_v, atr_v, conf_v, out_v,
             sem0, sem1, sem2, sem3, sem4):
    wid = lax.axis_index("s") * NC + lax.axis_index("c")
    base = wid * BPW
    cp_r = pltpu.async_copy(er_hbm, er_v, sem0)
    cp_a = pltpu.async_copy(ann_hbm.at[pl.ds(base, BPW)], ann_v, sem1)
    cp_t = pltpu.async_copy(atr_hbm.at[pl.ds(base, BPW)], atr_v, sem2)
    cp_c = pltpu.async_copy(conf_hbm.at[pl.ds(base, BPW)], conf_v, sem3)
    cp_m = pltpu.async_copy(ems_hbm, ems_v, sem4)
    pltpu.sync_copy(esp_hbm, esp_v)

    esp = [esp_v[i] for i in range(C * D)]

    cp_r.wait()
    cp_a.wait()
    cp_t.wait()
    cp_c.wait()
    cp_m.wait()

    @plsc.parallel_loop(0, NV, unroll=1)
    def body(j):
        off = j * L
        a_idx = ann_v[pl.ds(off, L)]
        t_idx = atr_v[pl.ds(off, L)]
        cvec = conf_v[pl.ds(off, L)]
        t4 = t_idx * D
        er = [plsc.load_gather(er_v, [t4 + d]) for d in range(D)]
        m = [a_idx == d for d in range(D - 1)]
        er_a = jnp.where(m[0], er[0],
                         jnp.where(m[1], er[1], jnp.where(m[2], er[2], er[3])))
        r_a = _ln(er_a)
        for c in range(C):
            s = er[0] * esp[c * D]
            for d in range(1, D):
                s = s + er[d] * esp[c * D + d]
            em_ca = plsc.load_gather(ems_v, [a_idx + (c * D)])
            ll = jnp.maximum((em_ca + r_a) - _ln(s), MIN_LL)
            out_v[c, pl.ds(off, L)] = cvec * ll

    pltpu.sync_copy(out_v, out_hbm.at[:, pl.ds(base, BPW)])


_sc_call = functools.partial(
    pl.kernel,
    out_type=jax.ShapeDtypeStruct((C, B), jnp.float32),
    mesh=plsc.VectorSubcoreMesh(core_axis_name="c", subcore_axis_name="s"),
    compiler_params=pltpu.CompilerParams(needs_layout_passes=False),
    scratch_types=[
        pltpu.VMEM((C * D, L), jnp.float32),
        pltpu.VMEM((C * D,), jnp.float32),
        pltpu.VMEM((V * D,), jnp.float32),
        pltpu.VMEM((BPW,), jnp.int32),
        pltpu.VMEM((BPW,), jnp.int32),
        pltpu.VMEM((BPW,), jnp.float32),
        pltpu.VMEM((C, BPW), jnp.float32),
        pltpu.SemaphoreType.DMA,
        pltpu.SemaphoreType.DMA,
        pltpu.SemaphoreType.DMA,
        pltpu.SemaphoreType.DMA,
        pltpu.SemaphoreType.DMA,
    ],
)(_sc_body)


def kernel(mu, random_effects, annotations, annotators, confidences):
    em = jnp.exp(mu)
    ems = (em - jnp.max(em, axis=1, keepdims=True)).reshape(C * D, 1)
    esp_splat = jnp.exp(jnp.broadcast_to(ems, (C * D, L)))
    return _sc_call(esp_splat, ems.reshape(C * D),
                    jnp.exp(random_effects.reshape(V * D)),
                    annotations.astype(jnp.int32),
                    annotators.astype(jnp.int32), confidences)


# speedup vs baseline: 1.0643x; 1.0643x over previous
import functools

import jax
import jax.numpy as jnp
from jax import lax
from jax.experimental import pallas as pl
from jax.experimental.pallas import tpu as pltpu
from jax.experimental.pallas import tpu_sc as plsc

C = 8
D = 4
V = 1000
B = 16384
L = 16
NC = 2
NS = 16
NW = NC * NS
BPW = B // NW
NV = BPW // L

LN2 = 0.6931471805599453
MIN_LL = -23.025850929940457
_P = [c * LN2 for c in (-2.153433788869514, 3.0475643969438027,
                        -1.051706410441485, 0.15822203552974923)]
_P0 = _P[0] - 127.0 * LN2


def _ln(s):
    bits = lax.bitcast_convert_type(s, jnp.int32)
    ef = lax.shift_right_arithmetic(bits, 23).astype(jnp.float32)
    mbits = (bits & 0x007FFFFF) | 0x3F800000
    m = lax.bitcast_convert_type(mbits, jnp.float32)
    p = _P[3]
    p = p * m + _P[2]
    p = p * m + _P[1]
    p = p * m + _P0
    return ef * LN2 + p


def _sc_body(esp_hbm, er_hbm, ann_hbm, atr_hbm, conf_hbm, out_hbm,
             esp_v, er_v, ann_v, atr_v, conf_v, out_v,
             sem0, sem1, sem2, sem3):
    wid = lax.axis_index("s") * NC + lax.axis_index("c")
    base = wid * BPW
    cp_r = pltpu.async_copy(er_hbm, er_v, sem0)
    cp_a = pltpu.async_copy(ann_hbm.at[pl.ds(base, BPW)], ann_v, sem1)
    cp_t = pltpu.async_copy(atr_hbm.at[pl.ds(base, BPW)], atr_v, sem2)
    cp_c = pltpu.async_copy(conf_hbm.at[pl.ds(base, BPW)], conf_v, sem3)
    pltpu.sync_copy(esp_hbm, esp_v)

    esp = [esp_v[i] for i in range(C * D)]

    cp_r.wait()
    cp_a.wait()
    cp_t.wait()
    cp_c.wait()

    @plsc.parallel_loop(0, NV, unroll=1)
    def body(j):
        off = j * L
        a_idx = ann_v[pl.ds(off, L)]
        t_idx = atr_v[pl.ds(off, L)]
        cvec = conf_v[pl.ds(off, L)]
        t4 = t_idx * D
        er = [plsc.load_gather(er_v, [t4 + d]) for d in range(D)]
        m = [a_idx == d for d in range(D - 1)]
        for c in range(C):
            t = [er[d] * esp[c * D + d] for d in range(D)]
            s = (t[0] + t[1]) + (t[2] + t[3])
            num = jnp.where(m[0], t[0],
                            jnp.where(m[1], t[1], jnp.where(m[2], t[2], t[3])))
            ll = jnp.maximum(_ln(num / s), MIN_LL)
            out_v[c, pl.ds(off, L)] = cvec * ll

    pltpu.sync_copy(out_v, out_hbm.at[:, pl.ds(base, BPW)])


_sc_call = functools.partial(
    pl.kernel,
    out_type=jax.ShapeDtypeStruct((C, B), jnp.float32),
    mesh=plsc.VectorSubcoreMesh(core_axis_name="c", subcore_axis_name="s"),
    compiler_params=pltpu.CompilerParams(needs_layout_passes=False),
    scratch_types=[
        pltpu.VMEM((C * D, L), jnp.float32),
        pltpu.VMEM((V * D,), jnp.float32),
        pltpu.VMEM((BPW,), jnp.int32),
        pltpu.VMEM((BPW,), jnp.int32),
        pltpu.VMEM((BPW,), jnp.float32),
        pltpu.VMEM((C, BPW), jnp.float32),
        pltpu.SemaphoreType.DMA,
        pltpu.SemaphoreType.DMA,
        pltpu.SemaphoreType.DMA,
        pltpu.SemaphoreType.DMA,
    ],
)(_sc_body)


def kernel(mu, random_effects, annotations, annotators, confidences):
    em = jnp.exp(mu)
    ems = (em - jnp.max(em, axis=1, keepdims=True)).reshape(C * D, 1)
    esp_splat = jnp.exp(jnp.broadcast_to(ems, (C * D, L)))
    return _sc_call(esp_splat, jnp.exp(random_effects.reshape(V * D)),
                    annotations.astype(jnp.int32),
                    annotators.astype(jnp.int32), confidences)
